# Initial kernel scaffold; baseline (speedup 1.0000x reference)
import jax
import jax.numpy as jnp
from jax.experimental import pallas as pl


def kernel(inputs, bn1_gamma, bn1_beta, bn1_mean, bn1_var, g1_w1, g1_b1, g1_w2, g1_b2, g2_w1, g2_b1, g2_w2, g2_b2, d1_w, d1_b, d2_w, d2_b, d3_w, d3_b, bn2_gamma, bn2_beta, bn2_mean, bn2_var, out_w, out_b):
    raise NotImplementedError("write your pallas kernel here")


# speedup vs baseline: 16.6329x; 16.6329x over previous
import jax
import jax.numpy as jnp
from jax.experimental import pallas as pl

_B, _N, _F = 4, 2048, 6
_K = 8
_R = 256
_NT = _N // _R
_EPS = 1e-3


def _knn_body(x_ref, xt_ref, sc_ref, of_ref, scT_ref, ofT_ref,
              w1_ref, b1_ref, w2_ref, b2_ref,
              upd1_ref, knn_ref):
    t = pl.program_id(1)
    sc = sc_ref[...]
    of = of_ref[...]
    xb = x_ref[0] * sc + of
    feat = xb[:, 2:]
    xr = x_ref[0, pl.ds(t * _R, _R), :] * sc + of
    prx = xr[:, 0:1]
    pry = xr[:, 1:2]
    posc = xt_ref[0, 0:2, :] * scT_ref[0:2] + ofT_ref[0:2]
    pcx = posc[0:1, :]
    pcy = posc[1:2, :]
    d2 = (prx - pcx) ** 2 + (pry - pcy) ** 2
    col = jax.lax.broadcasted_iota(jnp.int32, (_R, _N), 1)
    acc = jnp.zeros((_R, _N), jnp.float32)
    big = jnp.float32(3.0e38)
    for k in range(_K + 1):
        m = jnp.min(d2, axis=1, keepdims=True)
        first = jnp.min(jnp.where(d2 == m, col, _N), axis=1, keepdims=True)
        oh = col == first
        if k > 0:
            acc = acc + oh.astype(jnp.float32)
            knn_ref[0, :, k - 1] = first[:, 0]
        if k < _K:
            d2 = jnp.where(oh, big, d2)
    agg = jax.lax.dot(acc, feat, preferred_element_type=jnp.float32) * 0.125
    h = jnp.maximum(
        jnp.dot(agg, w1_ref[...], preferred_element_type=jnp.float32)
        + b1_ref[...], 0.0)
    upd1_ref[0] = (jnp.dot(h, w2_ref[...], preferred_element_type=jnp.float32)
                   + b2_ref[...])


def _tail_body(xr_ref, sc_ref, of_ref, upd1_ref, knn_ref,
               g2w1_ref, g2b1_ref, g2w2_ref, g2b2_ref,
               wp_ref, wu1_ref, wu2_ref, d1b_ref,
               d2w_ref, d2b_ref, d3w_ref, d3b_ref,
               sc2_ref, of2_ref, ow_ref, ob_ref,
               out_ref):
    t = pl.program_id(1)
    xr = xr_ref[0] * sc_ref[...] + of_ref[...]
    pos = xr[:, 0:2]
    table = upd1_ref[0]
    col = jax.lax.broadcasted_iota(jnp.int32, (_R, _N), 1)
    kn = knn_ref[0]
    acc = jnp.zeros((_R, _N), jnp.float32)
    for k in range(_K):
        acc = acc + (col == kn[:, k:k + 1]).astype(jnp.float32)
    agg = jax.lax.dot(acc, table, preferred_element_type=jnp.float32) * 0.125
    h = jnp.maximum(
        jnp.dot(agg, g2w1_ref[...], preferred_element_type=jnp.float32)
        + g2b1_ref[...], 0.0)
    upd2 = (jnp.dot(h, g2w2_ref[...], preferred_element_type=jnp.float32)
            + g2b2_ref[...])
    u1r = upd1_ref[0, pl.ds(t * _R, _R), :]
    y = (jnp.dot(pos, wp_ref[...], preferred_element_type=jnp.float32)
         + jnp.dot(u1r, wu1_ref[...], preferred_element_type=jnp.float32)
         + jnp.dot(upd2, wu2_ref[...], preferred_element_type=jnp.float32)
         + d1b_ref[...])
    y = jnp.maximum(y, 0.0)
    y = jnp.maximum(
        jnp.dot(y, d2w_ref[...], preferred_element_type=jnp.float32)
        + d2b_ref[...], 0.0)
    y = jnp.maximum(
        jnp.dot(y, d3w_ref[...], preferred_element_type=jnp.float32)
        + d3b_ref[...], 0.0)
    y = y * sc2_ref[...] + of2_ref[...]
    out_ref[0] = (jnp.sum(y * ow_ref[...], axis=1, keepdims=True)
                  + ob_ref[...])


def _full(shape):
    nd = len(shape)
    return pl.BlockSpec(shape, lambda b, t, _n=nd: (0,) * _n)


def kernel(inputs, bn1_gamma, bn1_beta, bn1_mean, bn1_var,
           g1_w1, g1_b1, g1_w2, g1_b2,
           g2_w1, g2_b1, g2_w2, g2_b2,
           d1_w, d1_b, d2_w, d2_b, d3_w, d3_b,
           bn2_gamma, bn2_beta, bn2_mean, bn2_var,
           out_w, out_b):
    f32 = jnp.float32
    sc1 = bn1_gamma / jnp.sqrt(bn1_var + _EPS)
    of1 = bn1_beta - bn1_mean * sc1
    sc_row = sc1.reshape(1, _F)
    of_row = of1.reshape(1, _F)
    scT = sc1.reshape(_F, 1)
    ofT = of1.reshape(_F, 1)
    xt = jnp.transpose(inputs, (0, 2, 1))

    upd1, knn = pl.pallas_call(
        _knn_body,
        grid=(_B, _NT),
        in_specs=[
            pl.BlockSpec((1, _N, _F), lambda b, t: (b, 0, 0)),
            pl.BlockSpec((1, _F, _N), lambda b, t: (b, 0, 0)),
            _full((1, _F)), _full((1, _F)),
            _full((_F, 1)), _full((_F, 1)),
            _full((_F - 2, 32)), _full((1, 32)),
            _full((32, 32)), _full((1, 32)),
        ],
        out_specs=[
            pl.BlockSpec((1, _R, 32), lambda b, t: (b, t, 0)),
            pl.BlockSpec((1, _R, _K), lambda b, t: (b, t, 0)),
        ],
        out_shape=[jax.ShapeDtypeStruct((_B, _N, 32), f32),
                   jax.ShapeDtypeStruct((_B, _N, _K), jnp.int32)],
    )(inputs, xt, sc_row, of_row, scT, ofT,
      g1_w1, g1_b1.reshape(1, 32), g1_w2, g1_b2.reshape(1, 32))

    sc2 = bn2_gamma / jnp.sqrt(bn2_var + _EPS)
    of2 = bn2_beta - bn2_mean * sc2
    wp = d1_w[0:2] + d1_w[34:36]
    wu1 = d1_w[2:34]
    wu2 = d1_w[36:68]

    out = pl.pallas_call(
        _tail_body,
        grid=(_B, _NT),
        in_specs=[
            pl.BlockSpec((1, _R, _F), lambda b, t: (b, t, 0)),
            _full((1, _F)), _full((1, _F)),
            pl.BlockSpec((1, _N, 32), lambda b, t: (b, 0, 0)),
            pl.BlockSpec((1, _R, _K), lambda b, t: (b, t, 0)),
            _full((32, 32)), _full((1, 32)),
            _full((32, 32)), _full((1, 32)),
            _full((2, 128)), _full((32, 128)), _full((32, 128)),
            _full((1, 128)),
            _full((128, 64)), _full((1, 64)),
            _full((64, 32)), _full((1, 32)),
            _full((1, 32)), _full((1, 32)),
            _full((1, 32)), _full((1, 1)),
        ],
        out_specs=pl.BlockSpec((1, _R, 1), lambda b, t: (b, t, 0)),
        out_shape=jax.ShapeDtypeStruct((_B, _N, 1), f32),
    )(inputs, sc_row, of_row, upd1, knn,
      g2_w1, g2_b1.reshape(1, 32), g2_w2, g2_b2.reshape(1, 32),
      wp, wu1, wu2, d1_b.reshape(1, 128),
      d2_w, d2_b.reshape(1, 64), d3_w, d3_b.reshape(1, 32),
      sc2.reshape(1, 32), of2.reshape(1, 32),
      out_w.reshape(1, 32), out_b.reshape(1, 1))
    return out
